# baseline (speedup 1.0000x reference)
import functools

import jax
import jax.numpy as jnp
from jax import lax
from jax.experimental import pallas as pl
from jax.experimental.pallas import tpu as pltpu
from jax.experimental.pallas import tpu_sc as plsc

D = 64
ROW = 128
GROUP = 4
C = GROUP * ROW


def _info():
    info = plsc.get_sparse_core_info()
    return info, info.num_cores * info.num_subcores


def _make_transpose(vocab: int):
    info, nw = _info()
    nt = vocab // ROW
    tail = vocab - nt * ROW
    per_w = -(-nt // nw)
    n_pairs = (per_w - 1) // 2
    mesh = plsc.VectorSubcoreMesh(core_axis_name="c", subcore_axis_name="s")

    @functools.partial(
        pl.kernel,
        mesh=mesh,
        out_type=jax.ShapeDtypeStruct((vocab // 2, 2 * D), jnp.float32),
        scratch_types=[
            pltpu.VMEM((D, ROW), jnp.float32),
            pltpu.VMEM((D, ROW), jnp.float32),
            pltpu.VMEM((D, ROW), jnp.float32),
            pltpu.VMEM((D, ROW), jnp.float32),
            pltpu.VMEM((tail // 2, 2 * D), jnp.float32),
            pltpu.SemaphoreType.DMA,
            pltpu.SemaphoreType.DMA,
            pltpu.SemaphoreType.DMA,
            pltpu.SemaphoreType.DMA,
        ],
        compiler_params=pltpu.CompilerParams(
            use_tc_tiling_on_sc=True, needs_layout_passes=False),
    )
    def k(tt_hbm, tail_hbm, out_hbm, st0, st1, tr0, tr1, stt,
          si0, si1, so0, so1):
        wid = lax.axis_index("s") * info.num_cores + lax.axis_index("c")
        ramp = lax.iota(jnp.int32, 16)
        ridx = [ramp + 16 * g for g in range(4)]

        def stage(ct, st, si):
            pltpu.async_copy(tt_hbm.at[:, pl.ds(ct * ROW, ROW)], st, si)

        UNROLL = 16

        def transpose(st, tr, ncc):
            def tbody(c2, _):
                for u in range(UNROLL):
                    cc = c2 * UNROLL + u
                    k = lax.shift_right_logical(cc, 1)
                    base = lax.mul(lax.rem(cc, 2), D)
                    cidx = jnp.full((16,), 0, jnp.int32) + cc
                    for g in range(4):
                        tr[k, pl.ds(base + 16 * g, 16)] = plsc.load_gather(
                            st, [ridx[g], cidx])
                return ()

            lax.fori_loop(0, ncc // UNROLL, tbody, (), unroll=False)

        def half(s, st, tr, si, so, first, last, prefetch_pred=None):
            ct = wid + nw * s
            pltpu.make_async_copy(
                tt_hbm.at[:, pl.ds(ct * ROW, ROW)], st, si).wait()
            if not first:
                pltpu.make_async_copy(
                    tr, out_hbm.at[pl.ds(ct * D, D)], so).wait()
            transpose(st, tr, ROW)
            if not last:
                if prefetch_pred is None:
                    stage(ct + 2 * nw, st, si)
                else:
                    @pl.when(prefetch_pred)
                    def _():
                        stage(ct + 2 * nw, st, si)
            pltpu.async_copy(tr, out_hbm.at[pl.ds(ct * D, D)], so)

        @pl.when(wid == 0)
        def _():
            pltpu.async_copy(tail_hbm, stt, si0).wait()
            pltpu.async_copy(
                stt, out_hbm.at[pl.ds(nt * D, tail // 2)], so0).wait()

        stage(wid, st0, si0)
        stage(wid + nw, st1, si1)

        half(0, st0, tr0, si0, so0, first=True, last=False)
        half(1, st1, tr1, si1, so1, first=True, last=False)

        def body(p, _):
            half(2 * p, st0, tr0, si0, so0, first=False, last=False)
            half(2 * p + 1, st1, tr1, si1, so1, first=False, last=False)
            return ()

        lax.fori_loop(1, n_pairs - 1, body, (), unroll=False)

        s_g = 2 * n_pairs
        guard = wid + nw * s_g < nt
        half(s_g - 2, st0, tr0, si0, so0, first=False, last=False,
             prefetch_pred=guard)
        half(s_g - 1, st1, tr1, si1, so1, first=False, last=True)

        @pl.when(guard)
        def _():
            half(s_g, st0, tr0, si0, so0, first=False, last=True)

        pltpu.make_async_copy(tr0, out_hbm.at[pl.ds(0, D)], so0).wait()
        pltpu.make_async_copy(tr1, out_hbm.at[pl.ds(0, D)], so1).wait()

    return k


def _make_gather(n_rows: int):
    info, nw = _info()
    rows_per_w = n_rows // nw
    n_steps = rows_per_w // GROUP
    n_pairs = n_steps // 2
    mesh = plsc.VectorSubcoreMesh(core_axis_name="c", subcore_axis_name="s")

    @functools.partial(
        pl.kernel,
        mesh=mesh,
        out_type=jax.ShapeDtypeStruct((n_rows * ROW, 2 * D), jnp.float32),
        scratch_types=[
            pltpu.VMEM((GROUP, ROW), jnp.int32),
            pltpu.VMEM((GROUP, ROW), jnp.int32),
            pltpu.VMEM((C, D), jnp.float32),
            pltpu.VMEM((C, D), jnp.float32),
            pltpu.SemaphoreType.DMA,
            pltpu.SemaphoreType.DMA,
            pltpu.SemaphoreType.DMA,
            pltpu.SemaphoreType.DMA,
            pltpu.SemaphoreType.DMA,
            pltpu.SemaphoreType.DMA,
        ],
        compiler_params=pltpu.CompilerParams(use_tc_tiling_on_sc=False),
    )
    def k(table_hbm, idx_hbm, out_hbm, idx0, idx1, rows0, rows1,
          si0, si1, sg0, sg1, so0, so1):
        wid = lax.axis_index("s") * info.num_cores + lax.axis_index("c")
        start = wid * rows_per_w

        def load_idx(r, idx_v, si):
            pltpu.async_copy(idx_hbm.at[pl.ds(r, GROUP)], idx_v, si)

        def half(r, idx_v, rows_v, si, sg, so, first, last):
            dst = out_hbm.at[pl.ds(r * ROW, C), pl.ds(0, D)]
            pltpu.make_async_copy(idx_hbm.at[pl.ds(r, GROUP)], idx_v, si).wait()
            if not first:
                pltpu.make_async_copy(rows_v, dst, so).wait()
            cps = [
                pltpu.async_copy(
                    table_hbm.at[idx_v.at[j]],
                    rows_v.at[pl.ds(j * ROW, ROW)],
                    sg,
                )
                for j in range(GROUP)
            ]
            for cp in cps:
                cp.wait()
            if not last:
                load_idx(r + 2 * GROUP, idx_v, si)
            pltpu.async_copy(rows_v, dst, so)

        load_idx(start, idx0, si0)
        load_idx(start + GROUP, idx1, si1)

        half(start, idx0, rows0, si0, sg0, so0, first=True, last=False)
        half(start + GROUP, idx1, rows1, si1, sg1, so1, first=True, last=False)

        def body(p, _):
            r = start + 2 * p * GROUP
            half(r, idx0, rows0, si0, sg0, so0, first=False, last=False)
            half(r + GROUP, idx1, rows1, si1, sg1, so1, first=False, last=False)
            return ()

        lax.fori_loop(1, n_pairs - 1, body, (), unroll=False)

        r = start + 2 * (n_pairs - 1) * GROUP
        half(r, idx0, rows0, si0, sg0, so0, first=False, last=True)
        half(r + GROUP, idx1, rows1, si1, sg1, so1, first=False, last=True)

        pltpu.make_async_copy(
            rows0, out_hbm.at[pl.ds(0, C), pl.ds(0, D)], so0).wait()
        pltpu.make_async_copy(
            rows1, out_hbm.at[pl.ds(0, C), pl.ds(0, D)], so1).wait()

    return k


def kernel(token_tensor, table):
    b, l = token_tensor.shape
    vocab = table.shape[0]
    idx = token_tensor.reshape(-1, ROW)
    nt = vocab // ROW
    tailp = table[nt * ROW:].reshape(-1, 2 * D)
    t2 = _make_transpose(vocab)(table.T, tailp)
    tab_rm = t2.reshape(vocab, D)
    out2 = _make_gather(idx.shape[0])(tab_rm, idx)
    return out2[:, :D].reshape(b, l, D)


# speedup vs baseline: 1.6530x; 1.6530x over previous
import functools

import jax
import jax.numpy as jnp
from jax import lax
from jax.experimental import pallas as pl
from jax.experimental.pallas import tpu as pltpu
from jax.experimental.pallas import tpu_sc as plsc

D = 64
ROW = 128
GROUP = 4
C = GROUP * ROW


def _info():
    info = plsc.get_sparse_core_info()
    return info, info.num_cores * info.num_subcores


def _make_transpose(vocab: int):
    B = 1024
    grid = -(-vocab // B)

    def body(tt_ref, o_ref):
        x = tt_ref[...]
        y = x.T.reshape(B // 2, 2, D)
        o_ref[...] = jnp.concatenate([y[:, 0, :], y[:, 1, :]], axis=1)

    return pl.pallas_call(
        body,
        grid=(grid,),
        in_specs=[pl.BlockSpec((D, B), lambda i: (0, i))],
        out_specs=pl.BlockSpec((B // 2, 2 * D), lambda i: (i, 0)),
        out_shape=jax.ShapeDtypeStruct((vocab // 2, 2 * D), jnp.float32),
    )


def _make_gather(n_rows: int):
    info, nw = _info()
    rows_per_w = n_rows // nw
    n_steps = rows_per_w // GROUP
    n_pairs = n_steps // 2
    mesh = plsc.VectorSubcoreMesh(core_axis_name="c", subcore_axis_name="s")

    @functools.partial(
        pl.kernel,
        mesh=mesh,
        out_type=jax.ShapeDtypeStruct((n_rows * ROW, 2 * D), jnp.float32),
        scratch_types=[
            pltpu.VMEM((GROUP, ROW), jnp.int32),
            pltpu.VMEM((GROUP, ROW), jnp.int32),
            pltpu.VMEM((C, D), jnp.float32),
            pltpu.VMEM((C, D), jnp.float32),
            pltpu.SemaphoreType.DMA,
            pltpu.SemaphoreType.DMA,
            pltpu.SemaphoreType.DMA,
            pltpu.SemaphoreType.DMA,
            pltpu.SemaphoreType.DMA,
            pltpu.SemaphoreType.DMA,
        ],
        compiler_params=pltpu.CompilerParams(use_tc_tiling_on_sc=False),
    )
    def k(table_hbm, idx_hbm, out_hbm, idx0, idx1, rows0, rows1,
          si0, si1, sg0, sg1, so0, so1):
        wid = lax.axis_index("s") * info.num_cores + lax.axis_index("c")
        start = wid * rows_per_w

        def load_idx(r, idx_v, si):
            pltpu.async_copy(idx_hbm.at[pl.ds(r, GROUP)], idx_v, si)

        def half(r, idx_v, rows_v, si, sg, so, first, last):
            dst = out_hbm.at[pl.ds(r * ROW, C), pl.ds(0, D)]
            pltpu.make_async_copy(idx_hbm.at[pl.ds(r, GROUP)], idx_v, si).wait()
            if not first:
                pltpu.make_async_copy(rows_v, dst, so).wait()
            cps = [
                pltpu.async_copy(
                    table_hbm.at[idx_v.at[j]],
                    rows_v.at[pl.ds(j * ROW, ROW)],
                    sg,
                )
                for j in range(GROUP)
            ]
            for cp in cps:
                cp.wait()
            if not last:
                load_idx(r + 2 * GROUP, idx_v, si)
            pltpu.async_copy(rows_v, dst, so)

        load_idx(start, idx0, si0)
        load_idx(start + GROUP, idx1, si1)

        half(start, idx0, rows0, si0, sg0, so0, first=True, last=False)
        half(start + GROUP, idx1, rows1, si1, sg1, so1, first=True, last=False)

        def body(p, _):
            r = start + 2 * p * GROUP
            half(r, idx0, rows0, si0, sg0, so0, first=False, last=False)
            half(r + GROUP, idx1, rows1, si1, sg1, so1, first=False, last=False)
            return ()

        lax.fori_loop(1, n_pairs - 1, body, (), unroll=False)

        r = start + 2 * (n_pairs - 1) * GROUP
        half(r, idx0, rows0, si0, sg0, so0, first=False, last=True)
        half(r + GROUP, idx1, rows1, si1, sg1, so1, first=False, last=True)

        pltpu.make_async_copy(
            rows0, out_hbm.at[pl.ds(0, C), pl.ds(0, D)], so0).wait()
        pltpu.make_async_copy(
            rows1, out_hbm.at[pl.ds(0, C), pl.ds(0, D)], so1).wait()

    return k


def kernel(token_tensor, table):
    b, l = token_tensor.shape
    vocab = table.shape[0]
    idx = token_tensor.reshape(-1, ROW)
    t2 = _make_transpose(vocab)(table.T)
    tab_rm = t2.reshape(vocab, D)
    out2 = _make_gather(idx.shape[0])(tab_rm, idx)
    return out2[:, :D].reshape(b, l, D)


# speedup vs baseline: 1.9759x; 1.1953x over previous
import functools

import jax
import jax.numpy as jnp
from jax import lax
from jax.experimental import pallas as pl
from jax.experimental.pallas import tpu as pltpu
from jax.experimental.pallas import tpu_sc as plsc

D = 64
ROW = 128
GROUP = 4
C = GROUP * ROW


def _make_gather(n_rows: int):
    info = plsc.get_sparse_core_info()
    nw = info.num_cores * info.num_subcores
    rows_per_w = n_rows // nw
    n_steps = rows_per_w // GROUP
    n_pairs = n_steps // 2
    mesh = plsc.VectorSubcoreMesh(core_axis_name="c", subcore_axis_name="s")

    @functools.partial(
        pl.kernel,
        mesh=mesh,
        out_type=jax.ShapeDtypeStruct((n_rows * ROW, 2 * D), jnp.float32),
        scratch_types=[
            pltpu.VMEM((GROUP, ROW), jnp.int32),
            pltpu.VMEM((GROUP, ROW), jnp.int32),
            pltpu.VMEM((C, D), jnp.float32),
            pltpu.VMEM((C, D), jnp.float32),
            pltpu.SemaphoreType.DMA,
            pltpu.SemaphoreType.DMA,
            pltpu.SemaphoreType.DMA,
            pltpu.SemaphoreType.DMA,
            pltpu.SemaphoreType.DMA,
            pltpu.SemaphoreType.DMA,
        ],
        compiler_params=pltpu.CompilerParams(use_tc_tiling_on_sc=False),
    )
    def k(table_hbm, idx_hbm, out_hbm, idx0, idx1, rows0, rows1,
          si0, si1, sg0, sg1, so0, so1):
        wid = lax.axis_index("s") * info.num_cores + lax.axis_index("c")
        start = wid * rows_per_w

        def load_idx(r, idx_v, si):
            pltpu.async_copy(idx_hbm.at[pl.ds(r, GROUP)], idx_v, si)

        def half(r, idx_v, rows_v, si, sg, so, first, last):
            dst = out_hbm.at[pl.ds(r * ROW, C), pl.ds(0, D)]
            pltpu.make_async_copy(idx_hbm.at[pl.ds(r, GROUP)], idx_v, si).wait()
            if not first:
                pltpu.make_async_copy(rows_v, dst, so).wait()
            cps = [
                pltpu.async_copy(
                    table_hbm.at[idx_v.at[j]],
                    rows_v.at[pl.ds(j * ROW, ROW)],
                    sg,
                )
                for j in range(GROUP)
            ]
            for cp in cps:
                cp.wait()
            if not last:
                load_idx(r + 2 * GROUP, idx_v, si)
            pltpu.async_copy(rows_v, dst, so)

        load_idx(start, idx0, si0)
        load_idx(start + GROUP, idx1, si1)

        half(start, idx0, rows0, si0, sg0, so0, first=True, last=False)
        half(start + GROUP, idx1, rows1, si1, sg1, so1, first=True, last=False)

        def body(p, _):
            r = start + 2 * p * GROUP
            half(r, idx0, rows0, si0, sg0, so0, first=False, last=False)
            half(r + GROUP, idx1, rows1, si1, sg1, so1, first=False, last=False)
            return ()

        lax.fori_loop(1, n_pairs - 1, body, (), unroll=False)

        r = start + 2 * (n_pairs - 1) * GROUP
        half(r, idx0, rows0, si0, sg0, so0, first=False, last=True)
        half(r + GROUP, idx1, rows1, si1, sg1, so1, first=False, last=True)

        pltpu.make_async_copy(
            rows0, out_hbm.at[pl.ds(0, C), pl.ds(0, D)], so0).wait()
        pltpu.make_async_copy(
            rows1, out_hbm.at[pl.ds(0, C), pl.ds(0, D)], so1).wait()

    return k


def kernel(token_tensor, table):
    b, l = token_tensor.shape
    vocab = table.shape[0]
    idx = token_tensor.reshape(-1, ROW)
    out2 = _make_gather(idx.shape[0])(table, idx)
    return out2[:, :D].reshape(b, l, D)
